# baseline (speedup 1.0000x reference)
import functools

import jax
import jax.numpy as jnp
from jax import lax
from jax.experimental import pallas as pl
from jax.experimental.pallas import tpu as pltpu, tpu_sc as plsc

N = 100000
E = 1600000
G = 64
F_IN = 30
F_HID = 40

NC, NS = 2, 16
NP = 100352
RT = NP // NS
NCH = 5
FC = 8

SBE = 2000
CSB = 1000
ET = E // NS
ET2 = E // (2 * NS)
NB = ET // (2 * CSB)
NB2 = ET2 // (2 * CSB)

BN = 2048
NG = NP // BN
PS = NP // 16
PB = BN // 16
FBIG = NCH * 128
XL = 16 * F_IN

_mesh = plsc.VectorSubcoreMesh(
    core_axis_name="c", subcore_axis_name="s", num_cores=NC, num_subcores=NS)


@functools.partial(
    pl.kernel,
    out_type=jax.ShapeDtypeStruct((NC, NP), jnp.float32),
    mesh=_mesh,
    compiler_params=pltpu.CompilerParams(use_tc_tiling_on_sc=False),
    scratch_types=[
        pltpu.VMEM_SHARED((NP,), jnp.float32),
        pltpu.VMEM((SBE,), jnp.int32),
        pltpu.VMEM((SBE,), jnp.float32),
        pltpu.VMEM((RT,), jnp.float32),
        pltpu.SemaphoreType.DMA,
    ],
)
def _deg_kernel(ei, out, acc, idxb, ones, zbuf, sem):
    c = lax.axis_index("c")
    t = lax.axis_index("s")
    r0 = t * RT

    def z16(i, carry):
        zbuf[pl.ds(i * 16, 16)] = jnp.zeros((16,), jnp.float32)
        return carry
    lax.fori_loop(0, RT // 16, z16, 0)

    def o16(i, carry):
        ones[pl.ds(i * 16, 16)] = jnp.ones((16,), jnp.float32)
        return carry
    lax.fori_loop(0, SBE // 16, o16, 0)

    pltpu.sync_copy(zbuf, acc.at[pl.ds(r0, RT)])
    plsc.subcore_barrier()

    w0 = (c * NS + t) * ET2

    def blk(sb, carry):
        base = w0 + sb * SBE
        pltpu.sync_copy(ei.at[1, pl.ds(base, SBE)], idxb)
        pltpu.async_copy(ones, acc.at[idxb], sem, add=True).wait()
        return carry
    lax.fori_loop(0, ET2 // SBE, blk, 0)

    plsc.subcore_barrier()
    pltpu.sync_copy(acc.at[pl.ds(r0, RT)], out.at[c, pl.ds(r0, RT)])


@functools.partial(
    pl.kernel,
    out_type=jax.ShapeDtypeStruct((NCH + 1, NP, FC), jnp.float32),
    mesh=_mesh,
    compiler_params=pltpu.CompilerParams(use_tc_tiling_on_sc=False),
    scratch_types=[
        pltpu.VMEM_SHARED((NP, FC), jnp.float32),
        pltpu.VMEM_SHARED((NP, FC), jnp.float32),
        pltpu.VMEM((2, CSB), jnp.int32),
        pltpu.VMEM((2, CSB), jnp.int32),
        pltpu.VMEM((2, CSB, FC), jnp.float32),
        pltpu.SemaphoreType.DMA,
        pltpu.SemaphoreType.DMA,
    ],
)
def _conv_kernel(ych, ei, out, tab, acc, sidx, didx, rows, gsem, ssem):
    c = lax.axis_index("c")
    t = lax.axis_index("s")
    r0 = t * RT

    def edge_pair(base):
        pltpu.sync_copy(ei.at[0, pl.ds(base, CSB)], sidx.at[0])
        pltpu.sync_copy(ei.at[1, pl.ds(base, CSB)], didx.at[0])
        ga = pltpu.async_copy(tab.at[sidx.at[0]], rows.at[0], gsem)
        pltpu.sync_copy(ei.at[0, pl.ds(base + CSB, CSB)], sidx.at[1])
        pltpu.sync_copy(ei.at[1, pl.ds(base + CSB, CSB)], didx.at[1])
        gb = pltpu.async_copy(tab.at[sidx.at[1]], rows.at[1], gsem)
        ga.wait()
        sa = pltpu.async_copy(rows.at[0], acc.at[didx.at[0]], ssem, add=True)
        gb.wait()
        sb_ = pltpu.async_copy(rows.at[1], acc.at[didx.at[1]], ssem, add=True)
        sa.wait()
        sb_.wait()

    for p in range(3):
        if p < 2:
            q = p * NC + c
            oslot = q
            e0 = t * ET
            nblk = NB
        else:
            q = NCH - 1
            oslot = NCH - 1 + c
            e0 = c * (E // 2) + t * ET2
            nblk = NB2
        pltpu.sync_copy(ych.at[q, pl.ds(r0, RT)], tab.at[pl.ds(r0, RT)])
        pltpu.sync_copy(ych.at[q, pl.ds(r0, RT)], acc.at[pl.ds(r0, RT)])
        plsc.subcore_barrier()

        def blk(sb, carry):
            edge_pair(e0 + sb * 2 * CSB)
            return carry
        lax.fori_loop(0, nblk, blk, 0)

        plsc.subcore_barrier()
        pltpu.sync_copy(acc.at[pl.ds(r0, RT)], out.at[oslot, pl.ds(r0, RT)])
        plsc.subcore_barrier()


def _dis_packed(deg_ref):
    dis16 = lax.rsqrt(1.0 + deg_ref[0] + deg_ref[1])
    e_rows = lax.broadcasted_iota(jnp.int32, (16, 128), 0)
    e_lane = lax.broadcasted_iota(jnp.int32, (16, 128), 1)
    e16 = (e_rows == e_lane // 8).astype(jnp.float32)
    dis_p = jnp.dot(dis16, e16, preferred_element_type=jnp.float32)
    return jnp.concatenate([dis_p] * NCH, axis=1)


def _bias_packed(b_ref):
    parts = []
    for q in range(NCH):
        bq = b_ref[...][:, q * FC:(q + 1) * FC]
        parts.append(jnp.concatenate([bq] * 16, axis=1))
    return jnp.concatenate(parts, axis=1)


def _merge_s(s_ref, y_ref):
    s4 = s_ref[4] + s_ref[5] - y_ref[0]
    return jnp.concatenate([s_ref[0], s_ref[1], s_ref[2], s_ref[3], s4],
                           axis=1)


def _prep_body(x_ref, w_ref, deg_ref, y_ref):
    xw = jnp.dot(x_ref[...], w_ref[...], preferred_element_type=jnp.float32)
    y = xw * _dis_packed(deg_ref)
    for q in range(NCH):
        y_ref[q] = y[:, q * 128:(q + 1) * 128]


def _mid_body(s_ref, y4_ref, deg_ref, b_ref, w_ref, y_ref):
    s = _merge_s(s_ref, y4_ref)
    dis = _dis_packed(deg_ref)
    h = jnp.maximum(s * dis + _bias_packed(b_ref), 0.0)
    xw = jnp.dot(h, w_ref[...], preferred_element_type=jnp.float32)
    y = xw * dis
    for q in range(NCH):
        y_ref[q] = y[:, q * 128:(q + 1) * 128]


def _fin_body(s_ref, y4_ref, deg_ref, b_ref, batch_ref, sel_ref, red_ref,
              wl_ref, bl_ref, out_ref, msum):
    i = pl.program_id(0)

    @pl.when(i == 0)
    def _():
        msum[...] = jnp.zeros_like(msum)

    s = _merge_s(s_ref, y4_ref)
    dis = _dis_packed(deg_ref)
    h = jnp.maximum(s * dis + _bias_packed(b_ref), 0.0)
    haug = jnp.concatenate([h, jnp.ones((PB, 128), jnp.float32)], axis=1)
    b16 = batch_ref[...]
    bt = jnp.concatenate([b16] * G, axis=1)
    gl = lax.broadcasted_iota(jnp.int32, (PB, 16 * G), 1) // 16
    oh = (bt == gl).astype(jnp.float32)
    msum[...] += lax.dot_general(oh, haug, (((0,), (0,)), ((), ())),
                                 preferred_element_type=jnp.float32)

    @pl.when(i == NG - 1)
    def _():
        m = msum[...]
        rr = lax.broadcasted_iota(jnp.int32, (16 * G, FBIG + 128), 0) % 16
        rc = (lax.broadcasted_iota(jnp.int32, (16 * G, FBIG + 128), 1)
              % 128) // 8
        mm = m * (rr == rc).astype(jnp.float32)
        pp = jnp.dot(sel_ref[...], mm, preferred_element_type=jnp.float32)
        pk = jnp.dot(pp, red_ref[...], preferred_element_type=jnp.float32)
        sums = pk[:, :F_HID]
        cnt = pk[:, F_HID:F_HID + 1]
        pooled = sums / jnp.maximum(cnt, 1.0)
        out_ref[...] = jnp.dot(pooled, wl_ref[...],
                               preferred_element_type=jnp.float32) + bl_ref[...]


_S_SPEC = pl.BlockSpec((NCH + 1, PB, 128), lambda i: (0, i, 0))
_Y4_SPEC = pl.BlockSpec((1, PB, 128), lambda i: (NCH - 1, i, 0))
_DEG_SPEC = pl.BlockSpec((2, PB, 16), lambda i: (0, i, 0))


def _prep_call(xpk, w1big, deg3):
    return pl.pallas_call(
        _prep_body,
        grid=(NG,),
        in_specs=[pl.BlockSpec((PB, XL), lambda i: (i, 0)),
                  pl.BlockSpec((XL, FBIG), lambda i: (0, 0)),
                  _DEG_SPEC],
        out_specs=pl.BlockSpec((NCH, PB, 128), lambda i: (0, i, 0)),
        out_shape=jax.ShapeDtypeStruct((NCH, PS, 128), jnp.float32),
    )(xpk, w1big, deg3)


def _mid_call(spk, ypk, deg3, b, wbig):
    return pl.pallas_call(
        _mid_body,
        grid=(NG,),
        in_specs=[_S_SPEC, _Y4_SPEC, _DEG_SPEC,
                  pl.BlockSpec((1, F_HID), lambda i: (0, 0)),
                  pl.BlockSpec((FBIG, FBIG), lambda i: (0, 0))],
        out_specs=pl.BlockSpec((NCH, PB, 128), lambda i: (0, i, 0)),
        out_shape=jax.ShapeDtypeStruct((NCH, PS, 128), jnp.float32),
    )(spk, ypk, deg3, b, wbig)


def _fin_call(spk, ypk, deg3, b, batchpk, sel, red, Wlin, blin):
    return pl.pallas_call(
        _fin_body,
        grid=(NG,),
        in_specs=[_S_SPEC, _Y4_SPEC, _DEG_SPEC,
                  pl.BlockSpec((1, F_HID), lambda i: (0, 0)),
                  pl.BlockSpec((PB, 16), lambda i: (i, 0)),
                  pl.BlockSpec((G, 16 * G), lambda i: (0, 0)),
                  pl.BlockSpec((FBIG + 128, F_HID + FC), lambda i: (0, 0)),
                  pl.BlockSpec((F_HID, 2), lambda i: (0, 0)),
                  pl.BlockSpec((1, 2), lambda i: (0, 0))],
        out_specs=pl.BlockSpec((G, 2), lambda i: (0, 0)),
        out_shape=jax.ShapeDtypeStruct((G, 2), jnp.float32),
        scratch_shapes=[pltpu.VMEM((16 * G, FBIG + 128), jnp.float32)],
    )(spk, ypk, deg3, b, batchpk, sel, red, Wlin, blin)


def _expand_w(w):
    w4 = w.reshape(NCH, FC, NCH, FC)
    big = jnp.einsum('qfpg,rs->qrfpsg', w4, jnp.eye(16, dtype=jnp.float32))
    return big.reshape(FBIG, FBIG)


def _expand_w1(w1):
    w3 = w1.reshape(F_IN, NCH, FC)
    big = jnp.einsum('kqf,rs->rkqsf', w3, jnp.eye(16, dtype=jnp.float32))
    return big.reshape(XL, FBIG)


def kernel(x, edge_index, batch, W1, b1, W2, b2, W3, b3, Wlin, blin):
    xpk = jnp.zeros((NP, F_IN), jnp.float32).at[:N].set(x).reshape(PS, XL)
    ei = edge_index.astype(jnp.int32)
    batchpk = jnp.full((NP,), G, jnp.int32).at[:N].set(
        batch.astype(jnp.int32)).reshape(PS, 16)

    w1big = _expand_w1(W1)
    w2big = _expand_w(W2)
    w3big = _expand_w(W3)
    b1r = b1.reshape(1, F_HID)
    b2r = b2.reshape(1, F_HID)
    b3r = b3.reshape(1, F_HID)

    gi = jnp.arange(16 * G, dtype=jnp.int32)
    sel = (gi[None, :] // 16 ==
           jnp.arange(G, dtype=jnp.int32)[:, None]).astype(jnp.float32)
    li = jnp.arange(FBIG + 128, dtype=jnp.int32)
    col = jnp.where(li < FBIG, (li // 128) * FC + li % 8, F_HID + li % 8)
    red = (col[:, None] ==
           jnp.arange(F_HID + FC, dtype=jnp.int32)[None, :]).astype(
               jnp.float32)

    def to_sc(a):
        return a.reshape(NCH, NP, FC)

    def to_tc(a):
        return a.reshape(NCH + 1, PS, 128)

    deg2 = _deg_kernel(ei)
    deg3 = deg2.reshape(2, PS, 16)
    y1 = _prep_call(xpk, w1big, deg3)
    y1t = y1.reshape(NCH, PS, 128)
    s1 = _conv_kernel(to_sc(y1), ei)
    y2 = _mid_call(to_tc(s1), y1t, deg3, b1r, w2big)
    y2t = y2.reshape(NCH, PS, 128)
    s2 = _conv_kernel(to_sc(y2), ei)
    y3 = _mid_call(to_tc(s2), y2t, deg3, b2r, w3big)
    y3t = y3.reshape(NCH, PS, 128)
    s3 = _conv_kernel(to_sc(y3), ei)
    out = _fin_call(to_tc(s3), y3t, deg3, b3r, batchpk, sel, red, Wlin,
                    blin.reshape(1, 2))
    return out


# speedup vs baseline: 1.1234x; 1.1234x over previous
import functools

import jax
import jax.numpy as jnp
from jax import lax
from jax.experimental import pallas as pl
from jax.experimental.pallas import tpu as pltpu, tpu_sc as plsc

N = 100000
E = 1600000
G = 64
F_IN = 30
F_HID = 40

NC, NS = 2, 16
NP = 100352
RT = NP // NS
NCH = 5
FC = 8

SBE = 2000
ET = E // NS
ET2 = E // (2 * NS)
NB = ET // SBE
NB2 = ET2 // SBE

BN = 2048
NG = NP // BN
PS = NP // 16
PB = BN // 16
FBIG = NCH * 128
XL = 16 * F_IN

_mesh = plsc.VectorSubcoreMesh(
    core_axis_name="c", subcore_axis_name="s", num_cores=NC, num_subcores=NS)


@functools.partial(
    pl.kernel,
    out_type=jax.ShapeDtypeStruct((NC, NP), jnp.float32),
    mesh=_mesh,
    compiler_params=pltpu.CompilerParams(use_tc_tiling_on_sc=False),
    scratch_types=[
        pltpu.VMEM_SHARED((NP,), jnp.float32),
        pltpu.VMEM((SBE,), jnp.int32),
        pltpu.VMEM((SBE,), jnp.float32),
        pltpu.VMEM((RT,), jnp.float32),
        pltpu.SemaphoreType.DMA,
    ],
)
def _deg_kernel(ei, out, acc, idxb, ones, zbuf, sem):
    c = lax.axis_index("c")
    t = lax.axis_index("s")
    r0 = t * RT

    def z16(i, carry):
        zbuf[pl.ds(i * 16, 16)] = jnp.zeros((16,), jnp.float32)
        return carry
    lax.fori_loop(0, RT // 16, z16, 0)

    def o16(i, carry):
        ones[pl.ds(i * 16, 16)] = jnp.ones((16,), jnp.float32)
        return carry
    lax.fori_loop(0, SBE // 16, o16, 0)

    pltpu.sync_copy(zbuf, acc.at[pl.ds(r0, RT)])
    plsc.subcore_barrier()

    w0 = (c * NS + t) * ET2

    def blk(sb, carry):
        base = w0 + sb * SBE
        pltpu.sync_copy(ei.at[1, pl.ds(base, SBE)], idxb)
        pltpu.async_copy(ones, acc.at[idxb], sem, add=True).wait()
        return carry
    lax.fori_loop(0, ET2 // SBE, blk, 0)

    plsc.subcore_barrier()
    pltpu.sync_copy(acc.at[pl.ds(r0, RT)], out.at[c, pl.ds(r0, RT)])


@functools.partial(
    pl.kernel,
    out_type=jax.ShapeDtypeStruct((NCH + 1, NP, FC), jnp.float32),
    mesh=_mesh,
    compiler_params=pltpu.CompilerParams(use_tc_tiling_on_sc=False),
    scratch_types=[
        pltpu.VMEM_SHARED((NP, FC), jnp.float32),
        pltpu.VMEM_SHARED((NP, FC), jnp.float32),
        pltpu.VMEM((SBE,), jnp.int32),
        pltpu.VMEM((SBE,), jnp.int32),
        pltpu.VMEM((SBE, FC), jnp.float32),
        pltpu.SemaphoreType.DMA,
        pltpu.SemaphoreType.DMA,
    ],
)
def _conv_kernel(ych, ei, out, tab, acc, sidx, didx, rows, gsem, ssem):
    c = lax.axis_index("c")
    t = lax.axis_index("s")
    r0 = t * RT

    def edge_block(base):
        pltpu.sync_copy(ei.at[0, pl.ds(base, SBE)], sidx)
        pltpu.sync_copy(ei.at[1, pl.ds(base, SBE)], didx)
        pltpu.async_copy(tab.at[sidx], rows, gsem).wait()
        pltpu.async_copy(rows, acc.at[didx], ssem, add=True).wait()

    for p in range(3):
        if p < 2:
            q = p * NC + c
            oslot = q
            e0 = t * ET
            nblk = NB
        else:
            q = NCH - 1
            oslot = NCH - 1 + c
            e0 = c * (E // 2) + t * ET2
            nblk = NB2
        pltpu.sync_copy(ych.at[q, pl.ds(r0, RT)], tab.at[pl.ds(r0, RT)])
        pltpu.sync_copy(ych.at[q, pl.ds(r0, RT)], acc.at[pl.ds(r0, RT)])
        plsc.subcore_barrier()

        def blk(sb, carry):
            edge_block(e0 + sb * SBE)
            return carry
        lax.fori_loop(0, nblk, blk, 0)

        plsc.subcore_barrier()
        pltpu.sync_copy(acc.at[pl.ds(r0, RT)], out.at[oslot, pl.ds(r0, RT)])
        plsc.subcore_barrier()


def _dis_packed(deg_ref):
    dis16 = lax.rsqrt(1.0 + deg_ref[0] + deg_ref[1])
    e_rows = lax.broadcasted_iota(jnp.int32, (16, 128), 0)
    e_lane = lax.broadcasted_iota(jnp.int32, (16, 128), 1)
    e16 = (e_rows == e_lane // 8).astype(jnp.float32)
    dis_p = jnp.dot(dis16, e16, preferred_element_type=jnp.float32)
    return jnp.concatenate([dis_p] * NCH, axis=1)


def _bias_packed(b_ref):
    parts = []
    for q in range(NCH):
        bq = b_ref[...][:, q * FC:(q + 1) * FC]
        parts.append(jnp.concatenate([bq] * 16, axis=1))
    return jnp.concatenate(parts, axis=1)


def _merge_s(s_ref, y_ref):
    s4 = s_ref[4] + s_ref[5] - y_ref[0]
    return jnp.concatenate([s_ref[0], s_ref[1], s_ref[2], s_ref[3], s4],
                           axis=1)


def _prep_body(x_ref, w_ref, deg_ref, y_ref):
    xw = jnp.dot(x_ref[...], w_ref[...], preferred_element_type=jnp.float32)
    y = xw * _dis_packed(deg_ref)
    for q in range(NCH):
        y_ref[q] = y[:, q * 128:(q + 1) * 128]


def _mid_body(s_ref, y4_ref, deg_ref, b_ref, w_ref, y_ref):
    s = _merge_s(s_ref, y4_ref)
    dis = _dis_packed(deg_ref)
    h = jnp.maximum(s * dis + _bias_packed(b_ref), 0.0)
    xw = jnp.dot(h, w_ref[...], preferred_element_type=jnp.float32)
    y = xw * dis
    for q in range(NCH):
        y_ref[q] = y[:, q * 128:(q + 1) * 128]


def _fin_body(s_ref, y4_ref, deg_ref, b_ref, batch_ref, sel_ref, red_ref,
              wl_ref, bl_ref, out_ref, msum):
    i = pl.program_id(0)

    @pl.when(i == 0)
    def _():
        msum[...] = jnp.zeros_like(msum)

    s = _merge_s(s_ref, y4_ref)
    dis = _dis_packed(deg_ref)
    h = jnp.maximum(s * dis + _bias_packed(b_ref), 0.0)
    haug = jnp.concatenate([h, jnp.ones((PB, 128), jnp.float32)], axis=1)
    b16 = batch_ref[...]
    bt = jnp.concatenate([b16] * G, axis=1)
    gl = lax.broadcasted_iota(jnp.int32, (PB, 16 * G), 1) // 16
    oh = (bt == gl).astype(jnp.float32)
    msum[...] += lax.dot_general(oh, haug, (((0,), (0,)), ((), ())),
                                 preferred_element_type=jnp.float32)

    @pl.when(i == NG - 1)
    def _():
        m = msum[...]
        rr = lax.broadcasted_iota(jnp.int32, (16 * G, FBIG + 128), 0) % 16
        rc = (lax.broadcasted_iota(jnp.int32, (16 * G, FBIG + 128), 1)
              % 128) // 8
        mm = m * (rr == rc).astype(jnp.float32)
        pp = jnp.dot(sel_ref[...], mm, preferred_element_type=jnp.float32)
        pk = jnp.dot(pp, red_ref[...], preferred_element_type=jnp.float32)
        sums = pk[:, :F_HID]
        cnt = pk[:, F_HID:F_HID + 1]
        pooled = sums / jnp.maximum(cnt, 1.0)
        out_ref[...] = jnp.dot(pooled, wl_ref[...],
                               preferred_element_type=jnp.float32) + bl_ref[...]


_S_SPEC = pl.BlockSpec((NCH + 1, PB, 128), lambda i: (0, i, 0))
_Y4_SPEC = pl.BlockSpec((1, PB, 128), lambda i: (NCH - 1, i, 0))
_DEG_SPEC = pl.BlockSpec((2, PB, 16), lambda i: (0, i, 0))


def _prep_call(xpk, w1big, deg3):
    return pl.pallas_call(
        _prep_body,
        grid=(NG,),
        in_specs=[pl.BlockSpec((PB, XL), lambda i: (i, 0)),
                  pl.BlockSpec((XL, FBIG), lambda i: (0, 0)),
                  _DEG_SPEC],
        out_specs=pl.BlockSpec((NCH, PB, 128), lambda i: (0, i, 0)),
        out_shape=jax.ShapeDtypeStruct((NCH, PS, 128), jnp.float32),
    )(xpk, w1big, deg3)


def _mid_call(spk, ypk, deg3, b, wbig):
    return pl.pallas_call(
        _mid_body,
        grid=(NG,),
        in_specs=[_S_SPEC, _Y4_SPEC, _DEG_SPEC,
                  pl.BlockSpec((1, F_HID), lambda i: (0, 0)),
                  pl.BlockSpec((FBIG, FBIG), lambda i: (0, 0))],
        out_specs=pl.BlockSpec((NCH, PB, 128), lambda i: (0, i, 0)),
        out_shape=jax.ShapeDtypeStruct((NCH, PS, 128), jnp.float32),
    )(spk, ypk, deg3, b, wbig)


def _fin_call(spk, ypk, deg3, b, batchpk, sel, red, Wlin, blin):
    return pl.pallas_call(
        _fin_body,
        grid=(NG,),
        in_specs=[_S_SPEC, _Y4_SPEC, _DEG_SPEC,
                  pl.BlockSpec((1, F_HID), lambda i: (0, 0)),
                  pl.BlockSpec((PB, 16), lambda i: (i, 0)),
                  pl.BlockSpec((G, 16 * G), lambda i: (0, 0)),
                  pl.BlockSpec((FBIG + 128, F_HID + FC), lambda i: (0, 0)),
                  pl.BlockSpec((F_HID, 2), lambda i: (0, 0)),
                  pl.BlockSpec((1, 2), lambda i: (0, 0))],
        out_specs=pl.BlockSpec((G, 2), lambda i: (0, 0)),
        out_shape=jax.ShapeDtypeStruct((G, 2), jnp.float32),
        scratch_shapes=[pltpu.VMEM((16 * G, FBIG + 128), jnp.float32)],
    )(spk, ypk, deg3, b, batchpk, sel, red, Wlin, blin)


def _expand_w(w):
    w4 = w.reshape(NCH, FC, NCH, FC)
    big = jnp.einsum('qfpg,rs->qrfpsg', w4, jnp.eye(16, dtype=jnp.float32))
    return big.reshape(FBIG, FBIG)


def _expand_w1(w1):
    w3 = w1.reshape(F_IN, NCH, FC)
    big = jnp.einsum('kqf,rs->rkqsf', w3, jnp.eye(16, dtype=jnp.float32))
    return big.reshape(XL, FBIG)


def kernel(x, edge_index, batch, W1, b1, W2, b2, W3, b3, Wlin, blin):
    xpk = jnp.zeros((NP, F_IN), jnp.float32).at[:N].set(x).reshape(PS, XL)
    ei = edge_index.astype(jnp.int32)
    batchpk = jnp.full((NP,), G, jnp.int32).at[:N].set(
        batch.astype(jnp.int32)).reshape(PS, 16)

    w1big = _expand_w1(W1)
    w2big = _expand_w(W2)
    w3big = _expand_w(W3)
    b1r = b1.reshape(1, F_HID)
    b2r = b2.reshape(1, F_HID)
    b3r = b3.reshape(1, F_HID)

    gi = jnp.arange(16 * G, dtype=jnp.int32)
    sel = (gi[None, :] // 16 ==
           jnp.arange(G, dtype=jnp.int32)[:, None]).astype(jnp.float32)
    li = jnp.arange(FBIG + 128, dtype=jnp.int32)
    col = jnp.where(li < FBIG, (li // 128) * FC + li % 8, F_HID + li % 8)
    red = (col[:, None] ==
           jnp.arange(F_HID + FC, dtype=jnp.int32)[None, :]).astype(
               jnp.float32)

    def to_sc(a):
        return a.reshape(NCH, NP, FC)

    def to_tc(a):
        return a.reshape(NCH + 1, PS, 128)

    deg2 = _deg_kernel(ei)
    deg3 = deg2.reshape(2, PS, 16)
    y1 = _prep_call(xpk, w1big, deg3)
    y1t = y1.reshape(NCH, PS, 128)
    s1 = _conv_kernel(to_sc(y1), ei)
    y2 = _mid_call(to_tc(s1), y1t, deg3, b1r, w2big)
    y2t = y2.reshape(NCH, PS, 128)
    s2 = _conv_kernel(to_sc(y2), ei)
    y3 = _mid_call(to_tc(s2), y2t, deg3, b2r, w3big)
    y3t = y3.reshape(NCH, PS, 128)
    s3 = _conv_kernel(to_sc(y3), ei)
    out = _fin_call(to_tc(s3), y3t, deg3, b3r, batchpk, sel, red, Wlin,
                    blin.reshape(1, 2))
    return out


# speedup vs baseline: 1.1663x; 1.0382x over previous
import functools

import jax
import jax.numpy as jnp
from jax import lax
from jax.experimental import pallas as pl
from jax.experimental.pallas import tpu as pltpu, tpu_sc as plsc

N = 100000
E = 1600000
G = 64
F_IN = 30
F_HID = 40

NC, NS = 2, 16
NP = 100352
RT = NP // NS
NCH = 5
FC = 8

SBE = 2000
ET = E // NS
ET2 = E // (2 * NS)
NB = ET // SBE
NB2 = ET2 // SBE

BN = 3584
NG = NP // BN
PS = NP // 16
PB = BN // 16
FBIG = NCH * 128
XL = 16 * F_IN

_mesh = plsc.VectorSubcoreMesh(
    core_axis_name="c", subcore_axis_name="s", num_cores=NC, num_subcores=NS)


@functools.partial(
    pl.kernel,
    out_type=jax.ShapeDtypeStruct((NC, NP), jnp.float32),
    mesh=_mesh,
    compiler_params=pltpu.CompilerParams(use_tc_tiling_on_sc=False),
    scratch_types=[
        pltpu.VMEM_SHARED((NP,), jnp.float32),
        pltpu.VMEM((SBE,), jnp.int32),
        pltpu.VMEM((SBE,), jnp.float32),
        pltpu.VMEM((RT,), jnp.float32),
        pltpu.SemaphoreType.DMA,
    ],
)
def _deg_kernel(ei, out, acc, idxb, ones, zbuf, sem):
    c = lax.axis_index("c")
    t = lax.axis_index("s")
    r0 = t * RT

    def z16(i, carry):
        zbuf[pl.ds(i * 16, 16)] = jnp.zeros((16,), jnp.float32)
        return carry
    lax.fori_loop(0, RT // 16, z16, 0)

    def o16(i, carry):
        ones[pl.ds(i * 16, 16)] = jnp.ones((16,), jnp.float32)
        return carry
    lax.fori_loop(0, SBE // 16, o16, 0)

    pltpu.sync_copy(zbuf, acc.at[pl.ds(r0, RT)])
    plsc.subcore_barrier()

    w0 = (c * NS + t) * ET2

    def blk(sb, carry):
        base = w0 + sb * SBE
        pltpu.sync_copy(ei.at[1, pl.ds(base, SBE)], idxb)
        pltpu.async_copy(ones, acc.at[idxb], sem, add=True).wait()
        return carry
    lax.fori_loop(0, ET2 // SBE, blk, 0)

    plsc.subcore_barrier()
    pltpu.sync_copy(acc.at[pl.ds(r0, RT)], out.at[c, pl.ds(r0, RT)])


@functools.partial(
    pl.kernel,
    out_type=jax.ShapeDtypeStruct((NCH + 1, NP, FC), jnp.float32),
    mesh=_mesh,
    compiler_params=pltpu.CompilerParams(use_tc_tiling_on_sc=False),
    scratch_types=[
        pltpu.VMEM_SHARED((NP, FC), jnp.float32),
        pltpu.VMEM_SHARED((NP, FC), jnp.float32),
        pltpu.VMEM((SBE,), jnp.int32),
        pltpu.VMEM((SBE,), jnp.int32),
        pltpu.VMEM((SBE, FC), jnp.float32),
        pltpu.SemaphoreType.DMA,
        pltpu.SemaphoreType.DMA,
    ],
)
def _conv_kernel(ych, ei, out, tab, acc, sidx, didx, rows, gsem, ssem):
    c = lax.axis_index("c")
    t = lax.axis_index("s")
    r0 = t * RT

    def edge_block(base):
        pltpu.sync_copy(ei.at[0, pl.ds(base, SBE)], sidx)
        pltpu.sync_copy(ei.at[1, pl.ds(base, SBE)], didx)
        pltpu.async_copy(tab.at[sidx], rows, gsem).wait()
        pltpu.async_copy(rows, acc.at[didx], ssem, add=True).wait()

    for p in range(3):
        if p < 2:
            q = p * NC + c
            oslot = q
            e0 = t * ET
            nblk = NB
        else:
            q = NCH - 1
            oslot = NCH - 1 + c
            e0 = c * (E // 2) + t * ET2
            nblk = NB2
        pltpu.sync_copy(ych.at[q, pl.ds(r0, RT)], tab.at[pl.ds(r0, RT)])
        pltpu.sync_copy(ych.at[q, pl.ds(r0, RT)], acc.at[pl.ds(r0, RT)])
        plsc.subcore_barrier()

        def blk(sb, carry):
            edge_block(e0 + sb * SBE)
            return carry
        lax.fori_loop(0, nblk, blk, 0)

        plsc.subcore_barrier()
        pltpu.sync_copy(acc.at[pl.ds(r0, RT)], out.at[oslot, pl.ds(r0, RT)])
        plsc.subcore_barrier()


def _dis_packed(deg_ref):
    dis16 = lax.rsqrt(1.0 + deg_ref[0] + deg_ref[1])
    e_rows = lax.broadcasted_iota(jnp.int32, (16, 128), 0)
    e_lane = lax.broadcasted_iota(jnp.int32, (16, 128), 1)
    e16 = (e_rows == e_lane // 8).astype(jnp.float32)
    dis_p = jnp.dot(dis16, e16, preferred_element_type=jnp.float32)
    return jnp.concatenate([dis_p] * NCH, axis=1)


def _bias_packed(b_ref):
    parts = []
    for q in range(NCH):
        bq = b_ref[...][:, q * FC:(q + 1) * FC]
        parts.append(jnp.concatenate([bq] * 16, axis=1))
    return jnp.concatenate(parts, axis=1)


def _merge_s(s_ref, y_ref):
    s4 = s_ref[4] + s_ref[5] - y_ref[0]
    return jnp.concatenate([s_ref[0], s_ref[1], s_ref[2], s_ref[3], s4],
                           axis=1)


def _prep_body(x_ref, w_ref, deg_ref, y_ref):
    xw = jnp.dot(x_ref[...], w_ref[...], preferred_element_type=jnp.float32)
    y = xw * _dis_packed(deg_ref)
    for q in range(NCH):
        y_ref[q] = y[:, q * 128:(q + 1) * 128]


def _mid_body(s_ref, y4_ref, deg_ref, b_ref, w_ref, y_ref):
    s = _merge_s(s_ref, y4_ref)
    dis = _dis_packed(deg_ref)
    h = jnp.maximum(s * dis + _bias_packed(b_ref), 0.0)
    xw = jnp.dot(h, w_ref[...], preferred_element_type=jnp.float32)
    y = xw * dis
    for q in range(NCH):
        y_ref[q] = y[:, q * 128:(q + 1) * 128]


def _fin_body(s_ref, y4_ref, deg_ref, b_ref, batch_ref, sel_ref, red_ref,
              wl_ref, bl_ref, out_ref, msum):
    i = pl.program_id(0)

    @pl.when(i == 0)
    def _():
        msum[...] = jnp.zeros_like(msum)

    s = _merge_s(s_ref, y4_ref)
    dis = _dis_packed(deg_ref)
    h = jnp.maximum(s * dis + _bias_packed(b_ref), 0.0)
    haug = jnp.concatenate([h, jnp.ones((PB, 128), jnp.float32)], axis=1)
    b16 = batch_ref[...]
    bt = jnp.concatenate([b16] * G, axis=1)
    gl = lax.broadcasted_iota(jnp.int32, (PB, 16 * G), 1) // 16
    oh = (bt == gl).astype(jnp.float32)
    msum[...] += lax.dot_general(oh, haug, (((0,), (0,)), ((), ())),
                                 preferred_element_type=jnp.float32)

    @pl.when(i == NG - 1)
    def _():
        m = msum[...]
        rr = lax.broadcasted_iota(jnp.int32, (16 * G, FBIG + 128), 0) % 16
        rc = (lax.broadcasted_iota(jnp.int32, (16 * G, FBIG + 128), 1)
              % 128) // 8
        mm = m * (rr == rc).astype(jnp.float32)
        pp = jnp.dot(sel_ref[...], mm, preferred_element_type=jnp.float32)
        pk = jnp.dot(pp, red_ref[...], preferred_element_type=jnp.float32)
        sums = pk[:, :F_HID]
        cnt = pk[:, F_HID:F_HID + 1]
        pooled = sums / jnp.maximum(cnt, 1.0)
        out_ref[...] = jnp.dot(pooled, wl_ref[...],
                               preferred_element_type=jnp.float32) + bl_ref[...]


_S_SPEC = pl.BlockSpec((NCH + 1, PB, 128), lambda i: (0, i, 0))
_Y4_SPEC = pl.BlockSpec((1, PB, 128), lambda i: (NCH - 1, i, 0))
_DEG_SPEC = pl.BlockSpec((2, PB, 16), lambda i: (0, i, 0))


def _prep_call(xpk, w1big, deg3):
    return pl.pallas_call(
        _prep_body,
        grid=(NG,),
        in_specs=[pl.BlockSpec((PB, XL), lambda i: (i, 0)),
                  pl.BlockSpec((XL, FBIG), lambda i: (0, 0)),
                  _DEG_SPEC],
        out_specs=pl.BlockSpec((NCH, PB, 128), lambda i: (0, i, 0)),
        out_shape=jax.ShapeDtypeStruct((NCH, PS, 128), jnp.float32),
    )(xpk, w1big, deg3)


def _mid_call(spk, ypk, deg3, b, wbig):
    return pl.pallas_call(
        _mid_body,
        grid=(NG,),
        in_specs=[_S_SPEC, _Y4_SPEC, _DEG_SPEC,
                  pl.BlockSpec((1, F_HID), lambda i: (0, 0)),
                  pl.BlockSpec((FBIG, FBIG), lambda i: (0, 0))],
        out_specs=pl.BlockSpec((NCH, PB, 128), lambda i: (0, i, 0)),
        out_shape=jax.ShapeDtypeStruct((NCH, PS, 128), jnp.float32),
    )(spk, ypk, deg3, b, wbig)


def _fin_call(spk, ypk, deg3, b, batchpk, sel, red, Wlin, blin):
    return pl.pallas_call(
        _fin_body,
        grid=(NG,),
        in_specs=[_S_SPEC, _Y4_SPEC, _DEG_SPEC,
                  pl.BlockSpec((1, F_HID), lambda i: (0, 0)),
                  pl.BlockSpec((PB, 16), lambda i: (i, 0)),
                  pl.BlockSpec((G, 16 * G), lambda i: (0, 0)),
                  pl.BlockSpec((FBIG + 128, F_HID + FC), lambda i: (0, 0)),
                  pl.BlockSpec((F_HID, 2), lambda i: (0, 0)),
                  pl.BlockSpec((1, 2), lambda i: (0, 0))],
        out_specs=pl.BlockSpec((G, 2), lambda i: (0, 0)),
        out_shape=jax.ShapeDtypeStruct((G, 2), jnp.float32),
        scratch_shapes=[pltpu.VMEM((16 * G, FBIG + 128), jnp.float32)],
    )(spk, ypk, deg3, b, batchpk, sel, red, Wlin, blin)


def _expand_w(w):
    w4 = w.reshape(NCH, FC, NCH, FC)
    big = jnp.einsum('qfpg,rs->qrfpsg', w4, jnp.eye(16, dtype=jnp.float32))
    return big.reshape(FBIG, FBIG)


def _expand_w1(w1):
    w3 = w1.reshape(F_IN, NCH, FC)
    big = jnp.einsum('kqf,rs->rkqsf', w3, jnp.eye(16, dtype=jnp.float32))
    return big.reshape(XL, FBIG)


def kernel(x, edge_index, batch, W1, b1, W2, b2, W3, b3, Wlin, blin):
    xpk = jnp.zeros((NP, F_IN), jnp.float32).at[:N].set(x).reshape(PS, XL)
    ei = edge_index.astype(jnp.int32)
    batchpk = jnp.full((NP,), G, jnp.int32).at[:N].set(
        batch.astype(jnp.int32)).reshape(PS, 16)

    w1big = _expand_w1(W1)
    w2big = _expand_w(W2)
    w3big = _expand_w(W3)
    b1r = b1.reshape(1, F_HID)
    b2r = b2.reshape(1, F_HID)
    b3r = b3.reshape(1, F_HID)

    gi = jnp.arange(16 * G, dtype=jnp.int32)
    sel = (gi[None, :] // 16 ==
           jnp.arange(G, dtype=jnp.int32)[:, None]).astype(jnp.float32)
    li = jnp.arange(FBIG + 128, dtype=jnp.int32)
    col = jnp.where(li < FBIG, (li // 128) * FC + li % 8, F_HID + li % 8)
    red = (col[:, None] ==
           jnp.arange(F_HID + FC, dtype=jnp.int32)[None, :]).astype(
               jnp.float32)

    def to_sc(a):
        return a.reshape(NCH, NP, FC)

    def to_tc(a):
        return a.reshape(NCH + 1, PS, 128)

    deg2 = _deg_kernel(ei)
    deg3 = deg2.reshape(2, PS, 16)
    y1 = _prep_call(xpk, w1big, deg3)
    y1t = y1.reshape(NCH, PS, 128)
    s1 = _conv_kernel(to_sc(y1), ei)
    y2 = _mid_call(to_tc(s1), y1t, deg3, b1r, w2big)
    y2t = y2.reshape(NCH, PS, 128)
    s2 = _conv_kernel(to_sc(y2), ei)
    y3 = _mid_call(to_tc(s2), y2t, deg3, b2r, w3big)
    y3t = y3.reshape(NCH, PS, 128)
    s3 = _conv_kernel(to_sc(y3), ei)
    out = _fin_call(to_tc(s3), y3t, deg3, b3r, batchpk, sel, red, Wlin,
                    blin.reshape(1, 2))
    return out
